# baseline (speedup 1.0000x reference)
import functools

import jax
import jax.numpy as jnp
from jax import lax
from jax.experimental import pallas as pl
from jax.experimental.pallas import tpu as pltpu
from jax.experimental.pallas import tpu_sc as plsc

N = 50000
E = 800000
HID = 64
HH = 32
NCLS_ = 4
NGRAPH = 256
NC, NS, LANES = 2, 16, 16

_MESH = plsc.VectorSubcoreMesh(
    core_axis_name="c", subcore_axis_name="s", num_cores=NC, num_subcores=NS)

CH = 128
EP = 819200
QE = EP // 4
NCH_E = EP // CH
SUP = 10
NSUP = NCH_E // SUP
ITERS_S = NSUP // NS
NPAD = 8

GCH = 80
NCH_G = N // GCH
ITERS_G = -(-NCH_G // NS)

PCH = 1000
NCH_P = N // PCH
ITERS_P = -(-NCH_P // NS)

ROWS_T = N // NS
STG = 125
STG_N = ROWS_T // STG


def _zero_rows(ref, nrows):
    z = jnp.zeros((LANES,), jnp.float32)

    @pl.loop(0, nrows)
    def _(i):
        ref[i, pl.ds(0, LANES)] = z
        ref[i, pl.ds(LANES, LANES)] = z


def _emb_body(x_hbm, embA, embB, outA, outB, idx_v, row_v, sem):
    s = lax.axis_index("s")
    c = lax.axis_index("c")

    @pl.loop(0, ITERS_G)
    def _(j):
        k = s + NS * j

        @pl.when(k < NCH_G)
        def _():
            base = k * GCH
            pltpu.sync_copy(x_hbm.at[pl.ds(base, GCH)], idx_v)

            @pl.when(c == 0)
            def _():
                pltpu.async_copy(embA.at[idx_v], row_v, sem).wait()
                pltpu.sync_copy(row_v, outA.at[pl.ds(base, GCH)])

            @pl.when(c == 1)
            def _():
                pltpu.async_copy(embB.at[idx_v], row_v, sem).wait()
                pltpu.sync_copy(row_v, outB.at[pl.ds(base, GCH)])


_emb_call = pl.kernel(
    _emb_body,
    out_type=(jax.ShapeDtypeStruct((N, HH), jnp.float32),) * 2,
    mesh=_MESH,
    compiler_params=pltpu.CompilerParams(use_tc_tiling_on_sc=False),
    scratch_types=[
        pltpu.VMEM((GCH,), jnp.int32),
        pltpu.VMEM((GCH, HH), jnp.float32),
        pltpu.SemaphoreType.DMA,
    ],
)


def _conv_body(src2_hbm, dst2_hbm, hA, hB, eA, eB, outA, outB,
               sidxb, didxb, h0b, h1b, e0b, e1b, stage,
               sg0, sg1, se0, se1, agg_sh):
    s = lax.axis_index("s")
    c = lax.axis_index("c")

    _zero_rows(stage, STG)
    r0 = s * ROWS_T

    @pl.loop(0, STG_N)
    def _(t):
        pltpu.sync_copy(stage, agg_sh.at[pl.ds(r0 + t * STG, STG)])

    plsc.subcore_barrier()

    hb = (h0b, h1b)
    eb_ = (e0b, e1b)
    sg = (sg0, sg1)
    se = (se0, se1)

    ECH = CH // 4

    @pl.loop(0, ITERS_S)
    def _(m):
        k = s + NS * m
        cbase = k * SUP
        pltpu.sync_copy(src2_hbm.at[pl.ds(cbase, SUP)], sidxb)
        pltpu.sync_copy(dst2_hbm.at[pl.ds(cbase, SUP)], didxb)

        def _issue(b):
            p = b & 1
            ebs = pl.ds((cbase + b) * ECH, ECH)

            @pl.when(c == 0)
            def _():
                pltpu.async_copy(hA.at[sidxb.at[b]], hb[p], sg[p])
                pltpu.async_copy(eA.at[ebs], eb_[p], se[p])

            @pl.when(c == 1)
            def _():
                pltpu.async_copy(hB.at[sidxb.at[b]], hb[p], sg[p])
                pltpu.async_copy(eB.at[ebs], eb_[p], se[p])

        def _wait(b):
            p = b & 1
            pltpu.make_async_copy(hA.at[sidxb.at[b]], hb[p], sg[p]).wait()
            pltpu.make_async_copy(
                eA.at[pl.ds((cbase + b) * ECH, ECH)], eb_[p], se[p]).wait()

        _issue(0)
        for b in range(SUP):
            p = b & 1
            if b + 1 < SUP:
                _issue(b + 1)
            _wait(b)
            hbp = hb[p]
            ebp = eb_[p]

            @plsc.parallel_loop(0, CH, unroll=8)
            def _(i):
                r = i // 4
                o = (i % 4) * HH
                v0 = hbp[i, pl.ds(0, LANES)] + ebp[r, pl.ds(o, LANES)]
                hbp[i, pl.ds(0, LANES)] = jnp.maximum(v0, 0.0)
                v1 = (hbp[i, pl.ds(LANES, LANES)]
                      + ebp[r, pl.ds(o + LANES, LANES)])
                hbp[i, pl.ds(LANES, LANES)] = jnp.maximum(v1, 0.0)

            pltpu.sync_copy(hbp, agg_sh.at[didxb.at[b]], add=True)

    plsc.subcore_barrier()

    @pl.loop(0, STG_N)
    def _(t):
        off = t * STG
        pltpu.sync_copy(agg_sh.at[pl.ds(r0 + off, STG)], stage)

        @pl.when(c == 0)
        def _():
            pltpu.sync_copy(stage, outA.at[pl.ds(r0 + off, STG)])

        @pl.when(c == 1)
        def _():
            pltpu.sync_copy(stage, outB.at[pl.ds(r0 + off, STG)])


_conv_call = pl.kernel(
    _conv_body,
    out_type=(jax.ShapeDtypeStruct((N, HH), jnp.float32),) * 2,
    mesh=_MESH,
    compiler_params=pltpu.CompilerParams(use_tc_tiling_on_sc=False),
    scratch_types=[
        pltpu.VMEM((SUP, CH), jnp.int32),
        pltpu.VMEM((SUP, CH), jnp.int32),
        pltpu.VMEM((CH, HH), jnp.float32),
        pltpu.VMEM((CH, HH), jnp.float32),
        pltpu.VMEM((CH // 4, 4 * HH), jnp.float32),
        pltpu.VMEM((CH // 4, 4 * HH), jnp.float32),
        pltpu.VMEM((STG, HH), jnp.float32),
        pltpu.SemaphoreType.DMA,
        pltpu.SemaphoreType.DMA,
        pltpu.SemaphoreType.DMA,
        pltpu.SemaphoreType.DMA,
        pltpu.VMEM_SHARED((N + NPAD, HH), jnp.float32),
    ],
)


def _pool_body(batch_hbm, hA, hB, outA, outB,
               bv, rows, pool_l, red, obuf, pool_sh):
    s = lax.axis_index("s")
    c = lax.axis_index("c")

    _zero_rows(pool_l, NGRAPH)

    @pl.loop(0, ITERS_P)
    def _(j):
        k = s + NS * j

        @pl.when(k < NCH_P)
        def _():
            base = k * PCH
            pltpu.sync_copy(batch_hbm.at[pl.ds(base, PCH)], bv.at[pl.ds(0, PCH)])

            @pl.when(c == 0)
            def _():
                pltpu.sync_copy(hA.at[pl.ds(base, PCH)], rows)

            @pl.when(c == 1)
            def _():
                pltpu.sync_copy(hB.at[pl.ds(base, PCH)], rows)

            @pl.loop(0, PCH)
            def _(i):
                g = bv[pl.ds(i, LANES)][0]
                pool_l[g, pl.ds(0, LANES)] = jnp.maximum(
                    pool_l[g, pl.ds(0, LANES)], rows[i, pl.ds(0, LANES)])
                pool_l[g, pl.ds(LANES, LANES)] = jnp.maximum(
                    pool_l[g, pl.ds(LANES, LANES)], rows[i, pl.ds(LANES, LANES)])

    pltpu.sync_copy(pool_l, pool_sh.at[s])
    plsc.subcore_barrier()

    g0 = s * (NGRAPH // NS)
    GG = NGRAPH // NS

    @pl.loop(0, NS)
    def _(t):
        pltpu.sync_copy(pool_sh.at[t, pl.ds(g0, GG)], red.at[t])

    @pl.loop(0, GG)
    def _(g):
        obuf[g, pl.ds(0, LANES)] = red[0, g, pl.ds(0, LANES)]
        obuf[g, pl.ds(LANES, LANES)] = red[0, g, pl.ds(LANES, LANES)]

        @pl.loop(1, NS)
        def _(t):
            obuf[g, pl.ds(0, LANES)] = jnp.maximum(
                obuf[g, pl.ds(0, LANES)], red[t, g, pl.ds(0, LANES)])
            obuf[g, pl.ds(LANES, LANES)] = jnp.maximum(
                obuf[g, pl.ds(LANES, LANES)], red[t, g, pl.ds(LANES, LANES)])

    @pl.when(c == 0)
    def _():
        pltpu.sync_copy(obuf, outA.at[pl.ds(g0, GG)])

    @pl.when(c == 1)
    def _():
        pltpu.sync_copy(obuf, outB.at[pl.ds(g0, GG)])


_pool_call = pl.kernel(
    _pool_body,
    out_type=(jax.ShapeDtypeStruct((NGRAPH, HH), jnp.float32),) * 2,
    mesh=_MESH,
    compiler_params=pltpu.CompilerParams(use_tc_tiling_on_sc=False),
    scratch_types=[
        pltpu.VMEM((PCH + LANES,), jnp.int32),
        pltpu.VMEM((PCH, HH), jnp.float32),
        pltpu.VMEM((NGRAPH, HH), jnp.float32),
        pltpu.VMEM((NS, NGRAPH // NS, HH), jnp.float32),
        pltpu.VMEM((NGRAPH // NS, HH), jnp.float32),
        pltpu.VMEM_SHARED((NS, NGRAPH, HH), jnp.float32),
    ],
)


BR4 = 1024


def _enc_body(a0, a1, a2, a3, wA_ref, wB_ref, bA_ref, bB_ref, oA_ref, oB_ref):
    for q, aq in enumerate((a0, a1, a2, a3)):
        vA = lax.dot_general(aq[...], wA_ref[...],
                             (((0,), (0,)), ((), ())),
                             preferred_element_type=jnp.float32) + bA_ref[...]
        vB = lax.dot_general(aq[...], wB_ref[...],
                             (((0,), (0,)), ((), ())),
                             preferred_element_type=jnp.float32) + bB_ref[...]
        oA_ref[:, pl.ds(q * HH, HH)] = vA
        oB_ref[:, pl.ds(q * HH, HH)] = vB


def _mk_aspec(q):
    return pl.BlockSpec((3, BR4), lambda i, q=q: (0, q * (QE // BR4) + i))


_enc_call = pl.pallas_call(
    _enc_body,
    grid=(QE // BR4,),
    in_specs=[
        _mk_aspec(0), _mk_aspec(1), _mk_aspec(2), _mk_aspec(3),
        pl.BlockSpec((3, HH), lambda i: (0, 0)),
        pl.BlockSpec((3, HH), lambda i: (0, 0)),
        pl.BlockSpec((1, HH), lambda i: (0, 0)),
        pl.BlockSpec((1, HH), lambda i: (0, 0)),
    ],
    out_specs=[pl.BlockSpec((BR4, 4 * HH), lambda i: (i, 0))] * 2,
    out_shape=[jax.ShapeDtypeStruct((QE, 4 * HH), jnp.float32)] * 2,
)


BN = 2000


def _mlp_body(hA, hB, aA, aB, w1, b1, w2, b2, oA, oB):
    x = (jnp.concatenate([hA[...], hB[...]], axis=1)
         + jnp.concatenate([aA[...], aB[...]], axis=1))
    t = jnp.maximum(
        jnp.dot(x, w1[...], preferred_element_type=jnp.float32) + b1[...], 0.0)
    y = jnp.maximum(
        jnp.dot(t, w2[...], preferred_element_type=jnp.float32) + b2[...], 0.0)
    oA[...] = y[:, :HH]
    oB[...] = y[:, HH:]


_mlp_call = pl.pallas_call(
    _mlp_body,
    grid=(N // BN,),
    in_specs=[
        pl.BlockSpec((BN, HH), lambda i: (i, 0)),
        pl.BlockSpec((BN, HH), lambda i: (i, 0)),
        pl.BlockSpec((BN, HH), lambda i: (i, 0)),
        pl.BlockSpec((BN, HH), lambda i: (i, 0)),
        pl.BlockSpec((HID, HID), lambda i: (0, 0)),
        pl.BlockSpec((1, HID), lambda i: (0, 0)),
        pl.BlockSpec((HID, HID), lambda i: (0, 0)),
        pl.BlockSpec((1, HID), lambda i: (0, 0)),
    ],
    out_specs=[pl.BlockSpec((BN, HH), lambda i: (i, 0))] * 2,
    out_shape=[jax.ShapeDtypeStruct((N, HH), jnp.float32)] * 2,
)


def _cls_body(pA, pB, w_ref, b_ref, o_ref):
    p = jnp.concatenate([pA[...], pB[...]], axis=1)
    o_ref[...] = jnp.dot(p, w_ref[...],
                         preferred_element_type=jnp.float32) + b_ref[...]


_cls_call = pl.pallas_call(
    _cls_body,
    grid=(1,),
    in_specs=[
        pl.BlockSpec((NGRAPH, HH), lambda i: (0, 0)),
        pl.BlockSpec((NGRAPH, HH), lambda i: (0, 0)),
        pl.BlockSpec((HID, 128), lambda i: (0, 0)),
        pl.BlockSpec((1, 128), lambda i: (0, 0)),
    ],
    out_specs=pl.BlockSpec((NGRAPH, 128), lambda i: (0, 0)),
    out_shape=jax.ShapeDtypeStruct((NGRAPH, 128), jnp.float32),
)


@jax.jit
def kernel(x, edge_index, edge_attr, batch, emb, eW, eb,
           w1a, b1a, w1b, b1b, w2a, b2a, w2b, b2b, cW, cb):
    i32 = jnp.int32
    x = x.astype(i32)
    src = edge_index[0].astype(i32)
    dst = edge_index[1].astype(i32)
    batch = batch.astype(i32)

    embA = emb[:, :HH]
    embB = emb[:, HH:]

    aTp = jnp.pad(edge_attr.T, ((0, 0), (0, EP - E)))
    eA, eB = _enc_call(aTp, aTp, aTp, aTp,
                       eW[:, :HH], eW[:, HH:],
                       eb[:HH].reshape(1, HH), eb[HH:].reshape(1, HH))
    h0A, h0B = _emb_call(x, embA, embB)

    src2 = jnp.pad(src, (0, EP - E)).reshape(4, QE).T.reshape(NCH_E, CH)
    dst2 = jnp.pad(dst, (0, EP - E),
                   constant_values=N).reshape(4, QE).T.reshape(NCH_E, CH)
    agA, agB = _conv_call(src2, dst2, h0A, h0B, eA, eB)
    h1A, h1B = _mlp_call(h0A, h0B, agA, agB, w1a, b1a.reshape(1, HID),
                         w1b, b1b.reshape(1, HID))

    agA2, agB2 = _conv_call(src2, dst2, h1A, h1B, eA, eB)
    h2A, h2B = _mlp_call(h1A, h1B, agA2, agB2, w2a, b2a.reshape(1, HID),
                         w2b, b2b.reshape(1, HID))

    pA, pB = _pool_call(batch, h2A, h2B)

    cWp = jnp.pad(cW, ((0, 0), (0, 128 - NCLS_)))
    cbp = jnp.pad(cb, (0, 128 - NCLS_)).reshape(1, 128)
    logits = _cls_call(pA, pB, cWp, cbp)[:, :NCLS_]
    return logits


# speedup vs baseline: 1.0494x; 1.0494x over previous
import functools

import jax
import jax.numpy as jnp
from jax import lax
from jax.experimental import pallas as pl
from jax.experimental.pallas import tpu as pltpu
from jax.experimental.pallas import tpu_sc as plsc

N = 50000
E = 800000
HID = 64
HH = 32
NCLS_ = 4
NGRAPH = 256
NC, NS, LANES = 2, 16, 16
M4 = N // 4

_MESH = plsc.VectorSubcoreMesh(
    core_axis_name="c", subcore_axis_name="s", num_cores=NC, num_subcores=NS)

CH = 128
EP = 819200
QE = EP // 4
NCH_E = EP // CH
SUP = 10
NSUP = NCH_E // SUP
ITERS_S = NSUP // NS
NPAD = 8

GCH = 80
NCH_G = N // GCH
ITERS_G = -(-NCH_G // NS)

PCH = 1000
NCH_P = N // PCH
ITERS_P = -(-NCH_P // NS)

ROWS_T = N // NS
STG = 125
STG_N = ROWS_T // STG


def _zero_rows(ref, nrows):
    z = jnp.zeros((LANES,), jnp.float32)

    @pl.loop(0, nrows)
    def _(i):
        ref[i, pl.ds(0, LANES)] = z
        ref[i, pl.ds(LANES, LANES)] = z


def _emb_body(x_hbm, embA, embB, outA, outB, idx_v, row_v, sem):
    s = lax.axis_index("s")
    c = lax.axis_index("c")

    @pl.loop(0, ITERS_G)
    def _(j):
        k = s + NS * j

        @pl.when(k < NCH_G)
        def _():
            base = k * GCH
            pltpu.sync_copy(x_hbm.at[pl.ds(base, GCH)], idx_v)

            @pl.when(c == 0)
            def _():
                pltpu.async_copy(embA.at[idx_v], row_v, sem).wait()
                pltpu.sync_copy(row_v, outA.at[pl.ds(base, GCH)])

            @pl.when(c == 1)
            def _():
                pltpu.async_copy(embB.at[idx_v], row_v, sem).wait()
                pltpu.sync_copy(row_v, outB.at[pl.ds(base, GCH)])


_emb_call = pl.kernel(
    _emb_body,
    out_type=(jax.ShapeDtypeStruct((N, HH), jnp.float32),) * 2,
    mesh=_MESH,
    compiler_params=pltpu.CompilerParams(use_tc_tiling_on_sc=False),
    scratch_types=[
        pltpu.VMEM((GCH,), jnp.int32),
        pltpu.VMEM((GCH, HH), jnp.float32),
        pltpu.SemaphoreType.DMA,
    ],
)


def _conv_body(src2_hbm, dst2_hbm, hA, hB, eA, eB, outA, outB,
               sidxb, didxb, h0b, h1b, e0b, e1b, stage,
               sg0, sg1, se0, se1, agg_sh):
    s = lax.axis_index("s")
    c = lax.axis_index("c")

    _zero_rows(stage, STG)
    r0 = s * ROWS_T

    @pl.loop(0, STG_N)
    def _(t):
        pltpu.sync_copy(stage, agg_sh.at[pl.ds(r0 + t * STG, STG)])

    plsc.subcore_barrier()

    hb = (h0b, h1b)
    eb_ = (e0b, e1b)
    sg = (sg0, sg1)
    se = (se0, se1)

    ECH = CH // 4

    @pl.loop(0, ITERS_S)
    def _(m):
        k = s + NS * m
        cbase = k * SUP
        pltpu.sync_copy(src2_hbm.at[pl.ds(cbase, SUP)], sidxb)
        pltpu.sync_copy(dst2_hbm.at[pl.ds(cbase, SUP)], didxb)

        def _issue(b):
            p = b & 1
            ebs = pl.ds((cbase + b) * ECH, ECH)

            @pl.when(c == 0)
            def _():
                pltpu.async_copy(hA.at[sidxb.at[b]], hb[p], sg[p])
                pltpu.async_copy(eA.at[ebs], eb_[p], se[p])

            @pl.when(c == 1)
            def _():
                pltpu.async_copy(hB.at[sidxb.at[b]], hb[p], sg[p])
                pltpu.async_copy(eB.at[ebs], eb_[p], se[p])

        def _wait(b):
            p = b & 1
            pltpu.make_async_copy(hA.at[sidxb.at[b]], hb[p], sg[p]).wait()
            pltpu.make_async_copy(
                eA.at[pl.ds((cbase + b) * ECH, ECH)], eb_[p], se[p]).wait()

        _issue(0)
        for b in range(SUP):
            p = b & 1
            if b + 1 < SUP:
                _issue(b + 1)
            _wait(b)
            hbp = hb[p]
            ebp = eb_[p]

            @plsc.parallel_loop(0, CH, unroll=8)
            def _(i):
                r = i // 4
                o = (i % 4) * HH
                v0 = hbp[i, pl.ds(0, LANES)] + ebp[r, pl.ds(o, LANES)]
                hbp[i, pl.ds(0, LANES)] = jnp.maximum(v0, 0.0)
                v1 = (hbp[i, pl.ds(LANES, LANES)]
                      + ebp[r, pl.ds(o + LANES, LANES)])
                hbp[i, pl.ds(LANES, LANES)] = jnp.maximum(v1, 0.0)

            pltpu.sync_copy(hbp, agg_sh.at[didxb.at[b]], add=True)

    plsc.subcore_barrier()

    @pl.loop(0, STG_N)
    def _(t):
        off = t * STG
        pltpu.sync_copy(agg_sh.at[pl.ds(r0 + off, STG)], stage)

        @pl.when(c == 0)
        def _():
            pltpu.sync_copy(stage, outA.at[pl.ds(r0 + off, STG)])

        @pl.when(c == 1)
        def _():
            pltpu.sync_copy(stage, outB.at[pl.ds(r0 + off, STG)])


_conv_call = pl.kernel(
    _conv_body,
    out_type=(jax.ShapeDtypeStruct((N, HH), jnp.float32),) * 2,
    mesh=_MESH,
    compiler_params=pltpu.CompilerParams(use_tc_tiling_on_sc=False),
    scratch_types=[
        pltpu.VMEM((SUP, CH), jnp.int32),
        pltpu.VMEM((SUP, CH), jnp.int32),
        pltpu.VMEM((CH, HH), jnp.float32),
        pltpu.VMEM((CH, HH), jnp.float32),
        pltpu.VMEM((CH // 4, 4 * HH), jnp.float32),
        pltpu.VMEM((CH // 4, 4 * HH), jnp.float32),
        pltpu.VMEM((STG, HH), jnp.float32),
        pltpu.SemaphoreType.DMA,
        pltpu.SemaphoreType.DMA,
        pltpu.SemaphoreType.DMA,
        pltpu.SemaphoreType.DMA,
        pltpu.VMEM_SHARED((N + NPAD, HH), jnp.float32),
    ],
)


def _pool_body(batch_hbm, hA, hB, outA, outB,
               bv, rows, pool_l, red, obuf, pool_sh):
    s = lax.axis_index("s")
    c = lax.axis_index("c")

    _zero_rows(pool_l, NGRAPH)

    @pl.loop(0, ITERS_P)
    def _(j):
        k = s + NS * j

        @pl.when(k < NCH_P)
        def _():
            base = k * PCH
            pltpu.sync_copy(batch_hbm.at[pl.ds(base, PCH)], bv.at[pl.ds(0, PCH)])

            @pl.when(c == 0)
            def _():
                pltpu.sync_copy(hA.at[pl.ds(base, PCH)], rows)

            @pl.when(c == 1)
            def _():
                pltpu.sync_copy(hB.at[pl.ds(base, PCH)], rows)

            @pl.loop(0, PCH)
            def _(i):
                g = bv[pl.ds(i, LANES)][0]
                pool_l[g, pl.ds(0, LANES)] = jnp.maximum(
                    pool_l[g, pl.ds(0, LANES)], rows[i, pl.ds(0, LANES)])
                pool_l[g, pl.ds(LANES, LANES)] = jnp.maximum(
                    pool_l[g, pl.ds(LANES, LANES)], rows[i, pl.ds(LANES, LANES)])

    pltpu.sync_copy(pool_l, pool_sh.at[s])
    plsc.subcore_barrier()

    g0 = s * (NGRAPH // NS)
    GG = NGRAPH // NS

    @pl.loop(0, NS)
    def _(t):
        pltpu.sync_copy(pool_sh.at[t, pl.ds(g0, GG)], red.at[t])

    @pl.loop(0, GG)
    def _(g):
        obuf[g, pl.ds(0, LANES)] = red[0, g, pl.ds(0, LANES)]
        obuf[g, pl.ds(LANES, LANES)] = red[0, g, pl.ds(LANES, LANES)]

        @pl.loop(1, NS)
        def _(t):
            obuf[g, pl.ds(0, LANES)] = jnp.maximum(
                obuf[g, pl.ds(0, LANES)], red[t, g, pl.ds(0, LANES)])
            obuf[g, pl.ds(LANES, LANES)] = jnp.maximum(
                obuf[g, pl.ds(LANES, LANES)], red[t, g, pl.ds(LANES, LANES)])

    @pl.when(c == 0)
    def _():
        pltpu.sync_copy(obuf, outA.at[pl.ds(g0, GG)])

    @pl.when(c == 1)
    def _():
        pltpu.sync_copy(obuf, outB.at[pl.ds(g0, GG)])


_pool_call = pl.kernel(
    _pool_body,
    out_type=(jax.ShapeDtypeStruct((NGRAPH, HH), jnp.float32),) * 2,
    mesh=_MESH,
    compiler_params=pltpu.CompilerParams(use_tc_tiling_on_sc=False),
    scratch_types=[
        pltpu.VMEM((PCH + LANES,), jnp.int32),
        pltpu.VMEM((PCH, HH), jnp.float32),
        pltpu.VMEM((NGRAPH, HH), jnp.float32),
        pltpu.VMEM((NS, NGRAPH // NS, HH), jnp.float32),
        pltpu.VMEM((NGRAPH // NS, HH), jnp.float32),
        pltpu.VMEM_SHARED((NS, NGRAPH, HH), jnp.float32),
    ],
)


BR4 = 1024


def _enc_body(a0, a1, a2, a3, wA_ref, wB_ref, bA_ref, bB_ref, oA_ref, oB_ref):
    for q, aq in enumerate((a0, a1, a2, a3)):
        vA = lax.dot_general(aq[...], wA_ref[...],
                             (((0,), (0,)), ((), ())),
                             preferred_element_type=jnp.float32) + bA_ref[...]
        vB = lax.dot_general(aq[...], wB_ref[...],
                             (((0,), (0,)), ((), ())),
                             preferred_element_type=jnp.float32) + bB_ref[...]
        oA_ref[:, pl.ds(q * HH, HH)] = vA
        oB_ref[:, pl.ds(q * HH, HH)] = vB


def _mk_aspec(q):
    return pl.BlockSpec((3, BR4), lambda i, q=q: (0, q * (QE // BR4) + i))


_enc_call = pl.pallas_call(
    _enc_body,
    grid=(QE // BR4,),
    in_specs=[
        _mk_aspec(0), _mk_aspec(1), _mk_aspec(2), _mk_aspec(3),
        pl.BlockSpec((3, HH), lambda i: (0, 0)),
        pl.BlockSpec((3, HH), lambda i: (0, 0)),
        pl.BlockSpec((1, HH), lambda i: (0, 0)),
        pl.BlockSpec((1, HH), lambda i: (0, 0)),
    ],
    out_specs=[pl.BlockSpec((BR4, 4 * HH), lambda i: (i, 0))] * 2,
    out_shape=[jax.ShapeDtypeStruct((QE, 4 * HH), jnp.float32)] * 2,
)


BN = M4


def _mlp_body(hA, hB, aA, aB, w1, b1, w2, b2, oA, oB):
    for q in range(4):
        sl = pl.ds(q * HH, HH)
        xA = hA[:, sl] + aA[:, sl]
        xB = hB[:, sl] + aB[:, sl]
        t = jnp.maximum(
            lax.dot_general(xA, w1[pl.ds(0, HH), :], (((1,), (0,)), ((), ())),
                            preferred_element_type=jnp.float32)
            + lax.dot_general(xB, w1[pl.ds(HH, HH), :],
                              (((1,), (0,)), ((), ())),
                              preferred_element_type=jnp.float32)
            + b1[...], 0.0)
        y = jnp.maximum(
            jnp.dot(t, w2[...], preferred_element_type=jnp.float32)
            + b2[...], 0.0)
        oA[:, sl] = y[:, :HH]
        oB[:, sl] = y[:, HH:]


_mlp_call = pl.pallas_call(
    _mlp_body,
    grid=(M4 // BN,),
    in_specs=[
        pl.BlockSpec((BN, 4 * HH), lambda i: (i, 0)),
        pl.BlockSpec((BN, 4 * HH), lambda i: (i, 0)),
        pl.BlockSpec((BN, 4 * HH), lambda i: (i, 0)),
        pl.BlockSpec((BN, 4 * HH), lambda i: (i, 0)),
        pl.BlockSpec((HID, HID), lambda i: (0, 0)),
        pl.BlockSpec((1, HID), lambda i: (0, 0)),
        pl.BlockSpec((HID, HID), lambda i: (0, 0)),
        pl.BlockSpec((1, HID), lambda i: (0, 0)),
    ],
    out_specs=[pl.BlockSpec((BN, 4 * HH), lambda i: (i, 0))] * 2,
    out_shape=[jax.ShapeDtypeStruct((M4, 4 * HH), jnp.float32)] * 2,
)


def _cls_body(pA, pB, w_ref, b_ref, o_ref):
    p = jnp.concatenate([pA[...], pB[...]], axis=1)
    o_ref[...] = jnp.dot(p, w_ref[...],
                         preferred_element_type=jnp.float32) + b_ref[...]


_cls_call = pl.pallas_call(
    _cls_body,
    grid=(1,),
    in_specs=[
        pl.BlockSpec((NGRAPH, HH), lambda i: (0, 0)),
        pl.BlockSpec((NGRAPH, HH), lambda i: (0, 0)),
        pl.BlockSpec((HID, 128), lambda i: (0, 0)),
        pl.BlockSpec((1, 128), lambda i: (0, 0)),
    ],
    out_specs=pl.BlockSpec((NGRAPH, 128), lambda i: (0, 0)),
    out_shape=jax.ShapeDtypeStruct((NGRAPH, 128), jnp.float32),
)


@jax.jit
def kernel(x, edge_index, edge_attr, batch, emb, eW, eb,
           w1a, b1a, w1b, b1b, w2a, b2a, w2b, b2b, cW, cb):
    i32 = jnp.int32
    x = x.astype(i32)
    src = edge_index[0].astype(i32)
    dst = edge_index[1].astype(i32)
    batch = batch.astype(i32)

    embA = emb[:, :HH]
    embB = emb[:, HH:]

    aTp = jnp.pad(edge_attr.T, ((0, 0), (0, EP - E)))
    eA, eB = _enc_call(aTp, aTp, aTp, aTp,
                       eW[:, :HH], eW[:, HH:],
                       eb[:HH].reshape(1, HH), eb[HH:].reshape(1, HH))

    x_p = x.reshape(4, M4).T.reshape(-1)
    batch_p = batch.reshape(4, M4).T.reshape(-1)
    src = 4 * (src % M4) + src // M4
    dst = 4 * (dst % M4) + dst // M4

    h0A, h0B = _emb_call(x_p, embA, embB)

    src2 = jnp.pad(src, (0, EP - E)).reshape(4, QE).T.reshape(NCH_E, CH)
    dst2 = jnp.pad(dst, (0, EP - E),
                   constant_values=N).reshape(4, QE).T.reshape(NCH_E, CH)
    agA, agB = _conv_call(src2, dst2, h0A, h0B, eA, eB)
    h1Ap, h1Bp = _mlp_call(h0A.reshape(M4, 128), h0B.reshape(M4, 128),
                           agA.reshape(M4, 128), agB.reshape(M4, 128),
                           w1a, b1a.reshape(1, HID), w1b, b1b.reshape(1, HID))
    h1A = h1Ap.reshape(N, HH)
    h1B = h1Bp.reshape(N, HH)

    agA2, agB2 = _conv_call(src2, dst2, h1A, h1B, eA, eB)
    h2Ap, h2Bp = _mlp_call(h1Ap, h1Bp,
                           agA2.reshape(M4, 128), agB2.reshape(M4, 128),
                           w2a, b2a.reshape(1, HID), w2b, b2b.reshape(1, HID))

    pA, pB = _pool_call(batch_p, h2Ap.reshape(N, HH), h2Bp.reshape(N, HH))

    cWp = jnp.pad(cW, ((0, 0), (0, 128 - NCLS_)))
    cbp = jnp.pad(cb, (0, 128 - NCLS_)).reshape(1, 128)
    logits = _cls_call(pA, pB, cWp, cbp)[:, :NCLS_]
    return logits


# speedup vs baseline: 1.2621x; 1.2026x over previous
import functools

import jax
import jax.numpy as jnp
from jax import lax
from jax.experimental import pallas as pl
from jax.experimental.pallas import tpu as pltpu
from jax.experimental.pallas import tpu_sc as plsc

N = 50000
E = 800000
HID = 64
HH = 32
NCLS_ = 4
NGRAPH = 256
NC, NS, LANES = 2, 16, 16
M4 = N // 4

_MESH = plsc.VectorSubcoreMesh(
    core_axis_name="c", subcore_axis_name="s", num_cores=NC, num_subcores=NS)

CH = 128
EP = 819200
QE = EP // 4
NCH_E = EP // CH
SUP = 10
NSUP = NCH_E // SUP
ITERS_S = NSUP // NS
NPAD = 8

GCH = 80
NCH_G = N // GCH
ITERS_G = -(-NCH_G // NS)

PCH = 1000
NCH_P = N // PCH
ITERS_P = -(-NCH_P // NS)

ROWS_T = N // NS
STG = 125
STG_N = ROWS_T // STG


def _zero_rows(ref, nrows):
    z = jnp.zeros((LANES,), jnp.float32)

    @pl.loop(0, nrows)
    def _(i):
        ref[i, pl.ds(0, LANES)] = z
        ref[i, pl.ds(LANES, LANES)] = z


def _emb_body(x_hbm, embA, embB, outA, outB, idx_v, row_v, sem):
    s = lax.axis_index("s")
    c = lax.axis_index("c")

    @pl.loop(0, ITERS_G)
    def _(j):
        k = s + NS * j

        @pl.when(k < NCH_G)
        def _():
            base = k * GCH
            pltpu.sync_copy(x_hbm.at[pl.ds(base, GCH)], idx_v)

            @pl.when(c == 0)
            def _():
                pltpu.async_copy(embA.at[idx_v], row_v, sem).wait()
                pltpu.sync_copy(row_v, outA.at[pl.ds(base, GCH)])

            @pl.when(c == 1)
            def _():
                pltpu.async_copy(embB.at[idx_v], row_v, sem).wait()
                pltpu.sync_copy(row_v, outB.at[pl.ds(base, GCH)])


_emb_call = pl.kernel(
    _emb_body,
    out_type=(jax.ShapeDtypeStruct((N, HH), jnp.float32),) * 2,
    mesh=_MESH,
    compiler_params=pltpu.CompilerParams(use_tc_tiling_on_sc=False),
    scratch_types=[
        pltpu.VMEM((GCH,), jnp.int32),
        pltpu.VMEM((GCH, HH), jnp.float32),
        pltpu.SemaphoreType.DMA,
    ],
)


def _conv_body(src2_hbm, dst2_hbm, hA, hB, eA, eB, outA, outB,
               sidxb, didxb, h0b, h1b, e0b, e1b, stage,
               sg0, sg1, se0, se1, agg_sh):
    s = lax.axis_index("s")
    c = lax.axis_index("c")

    _zero_rows(stage, STG)
    r0 = s * ROWS_T

    @pl.loop(0, STG_N)
    def _(t):
        pltpu.sync_copy(stage, agg_sh.at[pl.ds(r0 + t * STG, STG)])

    plsc.subcore_barrier()

    hb = (h0b, h1b)
    eb_ = (e0b, e1b)
    sg = (sg0, sg1)
    se = (se0, se1)

    ECH = CH // 4

    @pl.loop(0, ITERS_S)
    def _(m):
        k = s + NS * m
        cbase = k * SUP
        pltpu.sync_copy(src2_hbm.at[pl.ds(cbase, SUP)], sidxb)
        pltpu.sync_copy(dst2_hbm.at[pl.ds(cbase, SUP)], didxb)

        def _issue(b):
            p = b & 1
            ebs = pl.ds((cbase + b) * ECH, ECH)

            @pl.when(c == 0)
            def _():
                pltpu.async_copy(hA.at[sidxb.at[b]], hb[p], sg[p])
                pltpu.async_copy(eA.at[ebs], eb_[p], se[p])

            @pl.when(c == 1)
            def _():
                pltpu.async_copy(hB.at[sidxb.at[b]], hb[p], sg[p])
                pltpu.async_copy(eB.at[ebs], eb_[p], se[p])

        def _wait(b):
            p = b & 1
            pltpu.make_async_copy(hA.at[sidxb.at[b]], hb[p], sg[p]).wait()
            pltpu.make_async_copy(
                eA.at[pl.ds((cbase + b) * ECH, ECH)], eb_[p], se[p]).wait()

        _issue(0)
        for b in range(SUP):
            p = b & 1
            if b + 1 < SUP:
                _issue(b + 1)
            _wait(b)
            hbp = hb[p]
            ebp = eb_[p]

            @plsc.parallel_loop(0, CH, unroll=8)
            def _(i):
                r = i // 4
                o = (i % 4) * HH
                v0 = hbp[i, pl.ds(0, LANES)] + ebp[r, pl.ds(o, LANES)]
                hbp[i, pl.ds(0, LANES)] = jnp.maximum(v0, 0.0)
                v1 = (hbp[i, pl.ds(LANES, LANES)]
                      + ebp[r, pl.ds(o + LANES, LANES)])
                hbp[i, pl.ds(LANES, LANES)] = jnp.maximum(v1, 0.0)

            pltpu.sync_copy(hbp, agg_sh.at[didxb.at[b]], add=True)

    plsc.subcore_barrier()

    @pl.loop(0, STG_N)
    def _(t):
        off = t * STG
        pltpu.sync_copy(agg_sh.at[pl.ds(r0 + off, STG)], stage)

        @pl.when(c == 0)
        def _():
            pltpu.sync_copy(stage, outA.at[pl.ds(r0 + off, STG)])

        @pl.when(c == 1)
        def _():
            pltpu.sync_copy(stage, outB.at[pl.ds(r0 + off, STG)])


_conv_call = pl.kernel(
    _conv_body,
    out_type=(jax.ShapeDtypeStruct((N, HH), jnp.float32),) * 2,
    mesh=_MESH,
    compiler_params=pltpu.CompilerParams(use_tc_tiling_on_sc=False),
    scratch_types=[
        pltpu.VMEM((SUP, CH), jnp.int32),
        pltpu.VMEM((SUP, CH), jnp.int32),
        pltpu.VMEM((CH, HH), jnp.float32),
        pltpu.VMEM((CH, HH), jnp.float32),
        pltpu.VMEM((CH // 4, 4 * HH), jnp.float32),
        pltpu.VMEM((CH // 4, 4 * HH), jnp.float32),
        pltpu.VMEM((STG, HH), jnp.float32),
        pltpu.SemaphoreType.DMA,
        pltpu.SemaphoreType.DMA,
        pltpu.SemaphoreType.DMA,
        pltpu.SemaphoreType.DMA,
        pltpu.VMEM_SHARED((N + NPAD, HH), jnp.float32),
    ],
)


def _pool_body(batch_hbm, hA, hB, outA, outB,
               bv, rows, pool_l, red, obuf, pool_sh):
    s = lax.axis_index("s")
    c = lax.axis_index("c")

    _zero_rows(pool_l, NGRAPH)

    @pl.loop(0, ITERS_P)
    def _(j):
        k = s + NS * j

        @pl.when(k < NCH_P)
        def _():
            base = k * PCH
            pltpu.sync_copy(batch_hbm.at[pl.ds(base, PCH)], bv.at[pl.ds(0, PCH)])

            @pl.when(c == 0)
            def _():
                pltpu.sync_copy(hA.at[pl.ds(base, PCH)], rows)

            @pl.when(c == 1)
            def _():
                pltpu.sync_copy(hB.at[pl.ds(base, PCH)], rows)

            @pl.loop(0, PCH)
            def _(i):
                g = bv[pl.ds(i, LANES)][0]
                pool_l[g, pl.ds(0, LANES)] = jnp.maximum(
                    pool_l[g, pl.ds(0, LANES)], rows[i, pl.ds(0, LANES)])
                pool_l[g, pl.ds(LANES, LANES)] = jnp.maximum(
                    pool_l[g, pl.ds(LANES, LANES)], rows[i, pl.ds(LANES, LANES)])

    pltpu.sync_copy(pool_l, pool_sh.at[s])
    plsc.subcore_barrier()

    g0 = s * (NGRAPH // NS)
    GG = NGRAPH // NS

    @pl.loop(0, NS)
    def _(t):
        pltpu.sync_copy(pool_sh.at[t, pl.ds(g0, GG)], red.at[t])

    @pl.loop(0, GG)
    def _(g):
        obuf[g, pl.ds(0, LANES)] = red[0, g, pl.ds(0, LANES)]
        obuf[g, pl.ds(LANES, LANES)] = red[0, g, pl.ds(LANES, LANES)]

        @pl.loop(1, NS)
        def _(t):
            obuf[g, pl.ds(0, LANES)] = jnp.maximum(
                obuf[g, pl.ds(0, LANES)], red[t, g, pl.ds(0, LANES)])
            obuf[g, pl.ds(LANES, LANES)] = jnp.maximum(
                obuf[g, pl.ds(LANES, LANES)], red[t, g, pl.ds(LANES, LANES)])

    @pl.when(c == 0)
    def _():
        pltpu.sync_copy(obuf, outA.at[pl.ds(g0, GG)])

    @pl.when(c == 1)
    def _():
        pltpu.sync_copy(obuf, outB.at[pl.ds(g0, GG)])


_pool_call = pl.kernel(
    _pool_body,
    out_type=(jax.ShapeDtypeStruct((NGRAPH, HH), jnp.float32),) * 2,
    mesh=_MESH,
    compiler_params=pltpu.CompilerParams(use_tc_tiling_on_sc=False),
    scratch_types=[
        pltpu.VMEM((PCH + LANES,), jnp.int32),
        pltpu.VMEM((PCH, HH), jnp.float32),
        pltpu.VMEM((NGRAPH, HH), jnp.float32),
        pltpu.VMEM((NS, NGRAPH // NS, HH), jnp.float32),
        pltpu.VMEM((NGRAPH // NS, HH), jnp.float32),
        pltpu.VMEM_SHARED((NS, NGRAPH, HH), jnp.float32),
    ],
)


BR4 = 1024


def _enc_body(a_ref, w_ref, b_ref, oA_ref, oB_ref):
    v = lax.dot_general(a_ref[...], w_ref[...],
                        (((0,), (0,)), ((), ())),
                        preferred_element_type=jnp.float32) + b_ref[...]
    v4 = v.reshape(BR4, 4, HID)
    oA_ref[...] = v4[:, :, :HH].reshape(BR4, 4 * HH)
    oB_ref[...] = v4[:, :, HH:].reshape(BR4, 4 * HH)


_enc_call = pl.pallas_call(
    _enc_body,
    grid=(EP // (4 * BR4),),
    in_specs=[
        pl.BlockSpec((3, 4 * BR4), lambda i: (0, i)),
        pl.BlockSpec((3, HID), lambda i: (0, 0)),
        pl.BlockSpec((1, HID), lambda i: (0, 0)),
    ],
    out_specs=[pl.BlockSpec((BR4, 4 * HH), lambda i: (i, 0))] * 2,
    out_shape=[jax.ShapeDtypeStruct((EP // 4, 4 * HH), jnp.float32)] * 2,
)


BN = M4


def _mlp_body(hA, hB, aA, aB, w1, b1, w2, b2, oA, oB):
    for q in range(4):
        sl = pl.ds(q * HH, HH)
        xA = hA[:, sl] + aA[:, sl]
        xB = hB[:, sl] + aB[:, sl]
        t = jnp.maximum(
            lax.dot_general(xA, w1[pl.ds(0, HH), :], (((1,), (0,)), ((), ())),
                            preferred_element_type=jnp.float32)
            + lax.dot_general(xB, w1[pl.ds(HH, HH), :],
                              (((1,), (0,)), ((), ())),
                              preferred_element_type=jnp.float32)
            + b1[...], 0.0)
        y = jnp.maximum(
            jnp.dot(t, w2[...], preferred_element_type=jnp.float32)
            + b2[...], 0.0)
        oA[:, sl] = y[:, :HH]
        oB[:, sl] = y[:, HH:]


_mlp_call = pl.pallas_call(
    _mlp_body,
    grid=(M4 // BN,),
    in_specs=[
        pl.BlockSpec((BN, 4 * HH), lambda i: (i, 0)),
        pl.BlockSpec((BN, 4 * HH), lambda i: (i, 0)),
        pl.BlockSpec((BN, 4 * HH), lambda i: (i, 0)),
        pl.BlockSpec((BN, 4 * HH), lambda i: (i, 0)),
        pl.BlockSpec((HID, HID), lambda i: (0, 0)),
        pl.BlockSpec((1, HID), lambda i: (0, 0)),
        pl.BlockSpec((HID, HID), lambda i: (0, 0)),
        pl.BlockSpec((1, HID), lambda i: (0, 0)),
    ],
    out_specs=[pl.BlockSpec((BN, 4 * HH), lambda i: (i, 0))] * 2,
    out_shape=[jax.ShapeDtypeStruct((M4, 4 * HH), jnp.float32)] * 2,
)


def _cls_body(pA, pB, w_ref, b_ref, o_ref):
    p = jnp.concatenate([pA[...], pB[...]], axis=1)
    o_ref[...] = jnp.dot(p, w_ref[...],
                         preferred_element_type=jnp.float32) + b_ref[...]


_cls_call = pl.pallas_call(
    _cls_body,
    grid=(1,),
    in_specs=[
        pl.BlockSpec((NGRAPH, HH), lambda i: (0, 0)),
        pl.BlockSpec((NGRAPH, HH), lambda i: (0, 0)),
        pl.BlockSpec((HID, 128), lambda i: (0, 0)),
        pl.BlockSpec((1, 128), lambda i: (0, 0)),
    ],
    out_specs=pl.BlockSpec((NGRAPH, 128), lambda i: (0, 0)),
    out_shape=jax.ShapeDtypeStruct((NGRAPH, 128), jnp.float32),
)


@jax.jit
def kernel(x, edge_index, edge_attr, batch, emb, eW, eb,
           w1a, b1a, w1b, b1b, w2a, b2a, w2b, b2b, cW, cb):
    i32 = jnp.int32
    x = x.astype(i32)
    src = edge_index[0].astype(i32)
    dst = edge_index[1].astype(i32)
    batch = batch.astype(i32)

    embA = emb[:, :HH]
    embB = emb[:, HH:]

    aTp = jnp.pad(edge_attr.T, ((0, 0), (0, EP - E)))
    eA, eB = _enc_call(aTp, eW, eb.reshape(1, HID))

    h0A, h0B = _emb_call(x, embA, embB)

    src2 = jnp.pad(src, (0, EP - E)).reshape(NCH_E, CH)
    dst2 = jnp.pad(dst, (0, EP - E), constant_values=N).reshape(NCH_E, CH)
    agA, agB = _conv_call(src2, dst2, h0A, h0B, eA, eB)
    h1Ap, h1Bp = _mlp_call(h0A.reshape(M4, 128), h0B.reshape(M4, 128),
                           agA.reshape(M4, 128), agB.reshape(M4, 128),
                           w1a, b1a.reshape(1, HID), w1b, b1b.reshape(1, HID))
    h1A = h1Ap.reshape(N, HH)
    h1B = h1Bp.reshape(N, HH)

    agA2, agB2 = _conv_call(src2, dst2, h1A, h1B, eA, eB)
    h2Ap, h2Bp = _mlp_call(h1Ap, h1Bp,
                           agA2.reshape(M4, 128), agB2.reshape(M4, 128),
                           w2a, b2a.reshape(1, HID), w2b, b2b.reshape(1, HID))

    pA, pB = _pool_call(batch, h2Ap.reshape(N, HH), h2Bp.reshape(N, HH))

    cWp = jnp.pad(cW, ((0, 0), (0, 128 - NCLS_)))
    cbp = jnp.pad(cb, (0, 128 - NCLS_)).reshape(1, 128)
    logits = _cls_call(pA, pB, cWp, cbp)[:, :NCLS_]
    return logits
